# baseline (speedup 1.0000x reference)
import jax
import jax.numpy as jnp
from jax import lax
from jax.experimental import pallas as pl
from jax.experimental.pallas import tpu as pltpu
from jax.experimental.pallas import tpu_sc as plsc

N_FEAT = 26
IN_DIM = 101
OUT_D = 3
BATCH = 16384
OUT_W = N_FEAT * OUT_D
TAB_W = N_FEAT * IN_DIM * OUT_D

_NC, _NS, _L = 2, 16, 16
_NW = _NC * _NS
_BPW = BATCH // _NW
_NBLK = _BPW // _L
_HALF = _NBLK // 2


def _embed_body(idx_hbm, tab_hbm, out_hbm, idx_v, tab_v, out_v,
                sem_in, sem_out):
    wid = lax.axis_index("s") * _NC + lax.axis_index("c")
    base = wid * _BPW
    idx_cp = pltpu.async_copy(idx_hbm.at[:, pl.ds(base, _BPW)], idx_v, sem_in)
    pltpu.sync_copy(tab_hbm, tab_v)
    idx_cp.wait()

    def block(bb):
        col = bb * _L
        for f in range(N_FEAT):
            iv = idx_v[f, pl.ds(col, _L)]
            tbase = iv * OUT_D + f * (IN_DIM * OUT_D)
            for d in range(OUT_D):
                val = plsc.load_gather(tab_v, [tbase + d])
                out_v[f * OUT_D + d, pl.ds(col, _L)] = val

    plsc.parallel_loop(0, _HALF, 1)(block)
    cp1 = pltpu.async_copy(out_v.at[:, pl.ds(0, _HALF * _L)],
                           out_hbm.at[:, pl.ds(base, _HALF * _L)], sem_out)
    plsc.parallel_loop(_HALF, _NBLK, 1)(block)
    cp2 = pltpu.async_copy(out_v.at[:, pl.ds(_HALF * _L, _HALF * _L)],
                           out_hbm.at[:, pl.ds(base + _HALF * _L, _HALF * _L)],
                           sem_out)
    cp1.wait()
    cp2.wait()


def kernel(inputs, tables):
    call = pl.kernel(
        _embed_body,
        mesh=plsc.VectorSubcoreMesh(core_axis_name="c", subcore_axis_name="s"),
        out_type=jax.ShapeDtypeStruct((OUT_W, BATCH), jnp.float32),
        scratch_types=[
            pltpu.VMEM((N_FEAT, _BPW), jnp.int32),
            pltpu.VMEM((TAB_W,), jnp.float32),
            pltpu.VMEM((OUT_W, _BPW), jnp.float32),
            pltpu.SemaphoreType.DMA,
            pltpu.SemaphoreType.DMA,
        ],
        compiler_params=pltpu.CompilerParams(needs_layout_passes=False),
    )
    out_t = call(inputs.T, tables.reshape(-1))
    return out_t.T


# speedup vs baseline: 1.0967x; 1.0967x over previous
import jax
import jax.numpy as jnp
from jax import lax
from jax.experimental import pallas as pl
from jax.experimental.pallas import tpu as pltpu
from jax.experimental.pallas import tpu_sc as plsc

N_FEAT = 26
IN_DIM = 101
OUT_D = 3
BATCH = 16384
OUT_W = N_FEAT * OUT_D
TAB_W = N_FEAT * IN_DIM * OUT_D

_NC, _NS, _L = 2, 16, 16
_NW = _NC * _NS
_BPW = BATCH // _NW
_NBLK = _BPW // _L
_HALF = _NBLK // 2


def _embed_body(idx_hbm, tab_hbm, out_hbm, idx_v, tab_v, out_v,
                sem_in, sem_out):
    wid = lax.axis_index("s") * _NC + lax.axis_index("c")
    base = wid * _BPW
    idx_cp = pltpu.async_copy(idx_hbm.at[:, pl.ds(base, _BPW)], idx_v, sem_in)
    pltpu.sync_copy(tab_hbm, tab_v)
    idx_cp.wait()

    def block(bb):
        col = bb * _L
        for f in range(N_FEAT):
            iv = idx_v[f, pl.ds(col, _L)]
            tbase = iv * OUT_D + f * (IN_DIM * OUT_D)
            for d in range(OUT_D):
                val = plsc.load_gather(tab_v, [tbase + d])
                out_v[f * OUT_D + d, pl.ds(col, _L)] = val

    plsc.parallel_loop(0, _NBLK, 1)(block)
    pltpu.sync_copy(out_v, out_hbm.at[:, pl.ds(base, _BPW)])


def kernel(inputs, tables):
    call = pl.kernel(
        _embed_body,
        mesh=plsc.VectorSubcoreMesh(core_axis_name="c", subcore_axis_name="s"),
        out_type=jax.ShapeDtypeStruct((OUT_W, BATCH), jnp.float32),
        scratch_types=[
            pltpu.VMEM((N_FEAT, _BPW), jnp.int32),
            pltpu.VMEM((TAB_W,), jnp.float32),
            pltpu.VMEM((OUT_W, _BPW), jnp.float32),
            pltpu.SemaphoreType.DMA,
            pltpu.SemaphoreType.DMA,
        ],
        compiler_params=pltpu.CompilerParams(needs_layout_passes=False),
    )
    out_t = call(inputs.T, tables.reshape(-1))
    return out_t.T


# speedup vs baseline: 1.0993x; 1.0024x over previous
import jax
import jax.numpy as jnp
from jax import lax
from jax.experimental import pallas as pl
from jax.experimental.pallas import tpu as pltpu
from jax.experimental.pallas import tpu_sc as plsc

N_FEAT = 26
IN_DIM = 101
OUT_D = 3
BATCH = 16384
OUT_W = N_FEAT * OUT_D
TAB_W = N_FEAT * IN_DIM * OUT_D

_NC, _NS, _L = 2, 16, 16
_NW = _NC * _NS
_BPW = BATCH // _NW
_NBLK = _BPW // _L


def _embed_body(idx_hbm, tab_hbm, out_hbm, idx_v, tab_v, out_v,
                sem_in, sem_out):
    wid = lax.axis_index("s") * _NC + lax.axis_index("c")
    base = wid * _BPW
    idx_cp = pltpu.async_copy(idx_hbm.at[:, pl.ds(base, _BPW)], idx_v, sem_in)
    pltpu.sync_copy(tab_hbm, tab_v)
    idx_cp.wait()

    def block(bb):
        col = bb * _L
        for f in range(N_FEAT):
            iv = idx_v[f, pl.ds(col, _L)]
            tbase = iv * OUT_D + f * (IN_DIM * OUT_D)
            for d in range(OUT_D):
                val = plsc.load_gather(tab_v, [tbase + d])
                out_v[f * OUT_D + d, pl.ds(col, _L)] = val

    plsc.parallel_loop(0, _NBLK, 1)(block)
    pltpu.sync_copy(out_v, out_hbm.at[:, pl.ds(base, _BPW)])


def kernel(inputs, tables):
    call = pl.kernel(
        _embed_body,
        mesh=plsc.VectorSubcoreMesh(core_axis_name="c", subcore_axis_name="s"),
        out_type=jax.ShapeDtypeStruct((OUT_W, BATCH), jnp.float32),
        scratch_types=[
            pltpu.VMEM((N_FEAT, _BPW), jnp.int32),
            pltpu.VMEM((TAB_W,), jnp.float32),
            pltpu.VMEM((OUT_W, _BPW), jnp.float32),
            pltpu.SemaphoreType.DMA,
            pltpu.SemaphoreType.DMA,
        ],
        compiler_params=pltpu.CompilerParams(needs_layout_passes=False),
    )
    out_t = call(inputs.T, tables.reshape(-1))
    return out_t.T
